# Initial kernel scaffold; baseline (speedup 1.0000x reference)
import jax
import jax.numpy as jnp
from jax.experimental import pallas as pl


def kernel(x, edge_index, W1_l, b1_l, W1_r, W2_l, b2_l, W2_r):
    raise NotImplementedError("write your pallas kernel here")


# speedup vs baseline: 5.1767x; 5.1767x over previous
import functools

import jax
import jax.numpy as jnp
from jax import lax
from jax.experimental import pallas as pl
from jax.experimental.pallas import tpu as pltpu
from jax.experimental.pallas import tpu_sc as plsc

N = 10000
E = 320000
F = 128
NCORE = 2
NSUB = 16
NW = NCORE * NSUB
EPT = E // NW
CH = 80
NCHUNK = EPT // CH
NPAD = 10240
RPT = NPAD // NSUB
DCOL = 16


def _sc_agg(table, src, dst, *, with_deg):
  mesh = plsc.VectorSubcoreMesh(core_axis_name="c", subcore_axis_name="s")

  out_type = [jax.ShapeDtypeStruct((NCORE, NPAD, F), jnp.float32)]
  scratch = [
      pltpu.VMEM((CH,), jnp.int32),
      pltpu.VMEM((CH,), jnp.int32),
      pltpu.VMEM((CH, F), jnp.float32),
      pltpu.VMEM_SHARED((NPAD, F), jnp.float32),
      pltpu.SemaphoreType.DMA,
  ]
  if with_deg:
    out_type.append(jax.ShapeDtypeStruct((NCORE, NPAD, DCOL), jnp.float32))
    scratch += [
        pltpu.VMEM((CH, DCOL), jnp.float32),
        pltpu.VMEM_SHARED((NPAD, DCOL), jnp.float32),
    ]

  def body(table_hbm, src_hbm, dst_hbm, zrow_hbm, *rest):
    if with_deg:
      (zdeg_hbm, ones_hbm, acc_out, deg_out, src_v, dst_v, rows_v, acc_sh,
       sem, ones_v, deg_sh) = rest
    else:
      acc_out, src_v, dst_v, rows_v, acc_sh, sem = rest

    cid = lax.axis_index("c")
    sid = lax.axis_index("s")
    ebase = (cid * NSUB + sid) * EPT

    pltpu.sync_copy(zrow_hbm, acc_sh.at[pl.ds(sid * RPT, RPT)])
    if with_deg:
      pltpu.sync_copy(zdeg_hbm, deg_sh.at[pl.ds(sid * RPT, RPT)])
      pltpu.sync_copy(ones_hbm, ones_v)

    plsc.subcore_barrier()

    @pl.loop(0, NCHUNK)
    def _chunk(c):
      off = ebase + c * CH
      pltpu.sync_copy(src_hbm.at[pl.ds(off, CH)], src_v)
      pltpu.sync_copy(dst_hbm.at[pl.ds(off, CH)], dst_v)
      pltpu.async_copy(table_hbm.at[src_v], rows_v, sem).wait()
      pltpu.sync_copy(rows_v, acc_sh.at[dst_v], add=True)
      if with_deg:
        pltpu.sync_copy(ones_v, deg_sh.at[dst_v], add=True)

    plsc.subcore_barrier()

    pltpu.sync_copy(acc_sh.at[pl.ds(sid * RPT, RPT)],
                    acc_out.at[cid, pl.ds(sid * RPT, RPT)])
    if with_deg:
      pltpu.sync_copy(deg_sh.at[pl.ds(sid * RPT, RPT)],
                      deg_out.at[cid, pl.ds(sid * RPT, RPT)])

  run = pl.kernel(
      body, out_type=out_type, mesh=mesh, scratch_types=scratch,
      compiler_params=pltpu.CompilerParams(use_tc_tiling_on_sc=False))
  zrow = jnp.zeros((RPT, F), jnp.float32)
  if with_deg:
    res = run(table, src, dst, zrow,
              jnp.zeros((RPT, DCOL), jnp.float32),
              jnp.ones((CH, DCOL), jnp.float32))
  else:
    res = run(table, src, dst, zrow)
  return res if with_deg else res[0]


_RB = 1000
_GRID = N // _RB


def _row_spec():
  return pl.BlockSpec((_RB, F), lambda i: (i, 0))


def _full_spec():
  return pl.BlockSpec((F, F), lambda i: (0, 0))


def _deg_spec():
  return pl.BlockSpec((_RB, DCOL), lambda i: (i, 0))


def _pre_body(x_ref, wl_ref, wr_ref, b_ref, y_ref, r_ref):
  xb = x_ref[...]
  y_ref[...] = jnp.dot(xb, wl_ref[...], preferred_element_type=jnp.float32)
  r_ref[...] = (jnp.dot(xb, wr_ref[...], preferred_element_type=jnp.float32)
                + b_ref[...])


def _mid_body(p0, p1, d0, d1, r1, w2l, w2r, emb_ref, y2_ref, r2_ref):
  deg = jnp.maximum(d0[:, 0:1] + d1[:, 0:1], 1.0)
  emb = (p0[...] + p1[...]) / deg + r1[...]
  emb_ref[...] = emb
  h = jnp.maximum(emb, 0.0)
  y2_ref[...] = jnp.dot(h, w2l[...], preferred_element_type=jnp.float32)
  r2_ref[...] = jnp.dot(h, w2r[...], preferred_element_type=jnp.float32)


def _fin_body(q0, q1, d0, d1, r2, b2, out_ref):
  deg = jnp.maximum(d0[:, 0:1] + d1[:, 0:1], 1.0)
  out_ref[...] = (q0[...] + q1[...]) / deg + r2[...] + b2[...]


def kernel(x, edge_index, W1_l, b1_l, W1_r, W2_l, b2_l, W2_r):
  src = edge_index[0]
  dst = edge_index[1]
  f32 = jnp.float32

  pre = pl.pallas_call(
      _pre_body,
      grid=(_GRID,),
      in_specs=[_row_spec(), _full_spec(), _full_spec(),
                pl.BlockSpec((1, F), lambda i: (0, 0))],
      out_specs=[_row_spec(), _row_spec()],
      out_shape=[jax.ShapeDtypeStruct((N, F), f32)] * 2,
  )
  y1, r1 = pre(x, W1_l.T, W1_r.T, b1_l.reshape(1, F))

  acc1, degp = _sc_agg(y1, src, dst, with_deg=True)

  mid = pl.pallas_call(
      _mid_body,
      grid=(_GRID,),
      in_specs=[_row_spec(), _row_spec(), _deg_spec(), _deg_spec(),
                _row_spec(), _full_spec(), _full_spec()],
      out_specs=[_row_spec(), _row_spec(), _row_spec()],
      out_shape=[jax.ShapeDtypeStruct((N, F), f32)] * 3,
  )
  emb, y2, r2 = mid(acc1[0], acc1[1], degp[0], degp[1], r1,
                    W2_l.T, W2_r.T)

  acc2 = _sc_agg(y2, src, dst, with_deg=False)

  fin = pl.pallas_call(
      _fin_body,
      grid=(_GRID,),
      in_specs=[_row_spec(), _row_spec(), _deg_spec(), _deg_spec(),
                _row_spec(), pl.BlockSpec((1, F), lambda i: (0, 0))],
      out_specs=_row_spec(),
      out_shape=jax.ShapeDtypeStruct((N, F), f32),
  )
  logits = fin(acc2[0], acc2[1], degp[0], degp[1], r2, b2_l.reshape(1, F))

  return (logits, emb)


# speedup vs baseline: 9.0541x; 1.7490x over previous
import functools

import jax
import jax.numpy as jnp
from jax import lax
from jax.experimental import pallas as pl
from jax.experimental.pallas import tpu as pltpu
from jax.experimental.pallas import tpu_sc as plsc

N = 10000
E = 320000
F = 128
NCORE = 2
NSUB = 16
NW = NCORE * NSUB
EPT = E // NW
CH = 80
NCHUNK = EPT // CH
NPAD = 10240
RPT = NPAD // NSUB
DCOL = 16


def _sc_agg(table, src, dst, *, with_deg):
  mesh = plsc.VectorSubcoreMesh(core_axis_name="c", subcore_axis_name="s")

  out_type = [jax.ShapeDtypeStruct((NCORE, NPAD, F), jnp.float32)]
  scratch = [
      pltpu.VMEM((4, CH), jnp.int32),
      pltpu.VMEM((4, CH), jnp.int32),
      pltpu.VMEM((2, CH, F), jnp.float32),
      pltpu.VMEM_SHARED((NPAD, F), jnp.float32),
      pltpu.SemaphoreType.DMA((4,)),
      pltpu.SemaphoreType.DMA((2,)),
      pltpu.SemaphoreType.DMA((2,)),
  ]
  if with_deg:
    out_type.append(jax.ShapeDtypeStruct((NCORE, NPAD, DCOL), jnp.float32))
    scratch += [
        pltpu.VMEM((CH, DCOL), jnp.float32),
        pltpu.VMEM_SHARED((NPAD, DCOL), jnp.float32),
        pltpu.SemaphoreType.DMA((2,)),
    ]

  def body(table_hbm, src_hbm, dst_hbm, zrow_hbm, *rest):
    if with_deg:
      (zdeg_hbm, ones_hbm, acc_out, deg_out, src_v, dst_v, rows_v, acc_sh,
       sem_i, sem_g, sem_s, ones_v, deg_sh, sem_d) = rest
    else:
      (acc_out, src_v, dst_v, rows_v, acc_sh, sem_i, sem_g, sem_s) = rest

    cid = lax.axis_index("c")
    sid = lax.axis_index("s")
    ebase = (cid * NSUB + sid) * EPT

    def idx_start(c, q):
      off = ebase + c * CH
      pltpu.async_copy(src_hbm.at[pl.ds(off, CH)], src_v.at[q], sem_i.at[q])
      pltpu.async_copy(dst_hbm.at[pl.ds(off, CH)], dst_v.at[q], sem_i.at[q])

    def idx_wait(q):
      pltpu.make_async_copy(src_hbm.at[pl.ds(0, CH)], src_v.at[q],
                            sem_i.at[q]).wait()
      pltpu.make_async_copy(dst_hbm.at[pl.ds(0, CH)], dst_v.at[q],
                            sem_i.at[q]).wait()

    def gather_start(q, b):
      pltpu.async_copy(table_hbm.at[src_v.at[q]], rows_v.at[b], sem_g.at[b])

    def gather_wait(q, b):
      pltpu.make_async_copy(table_hbm.at[src_v.at[q]], rows_v.at[b],
                            sem_g.at[b]).wait()

    def scat_start(q, b):
      pltpu.async_copy(rows_v.at[b], acc_sh.at[dst_v.at[q]], sem_s.at[b],
                       add=True)
      if with_deg:
        pltpu.async_copy(ones_v, deg_sh.at[dst_v.at[q]], sem_d.at[b],
                         add=True)

    def scat_wait(q, b):
      pltpu.make_async_copy(rows_v.at[b], acc_sh.at[dst_v.at[q]],
                            sem_s.at[b]).wait()
      if with_deg:
        pltpu.make_async_copy(ones_v, deg_sh.at[dst_v.at[q]],
                              sem_d.at[b]).wait()

    pltpu.sync_copy(zrow_hbm, acc_sh.at[pl.ds(sid * RPT, RPT)])
    if with_deg:
      pltpu.sync_copy(zdeg_hbm, deg_sh.at[pl.ds(sid * RPT, RPT)])
      pltpu.sync_copy(ones_hbm, ones_v)

    plsc.subcore_barrier()

    idx_start(0, 0)
    idx_start(1, 1)
    idx_start(2, 2)
    idx_wait(0)
    gather_start(0, 0)

    gather_wait(0, 0)
    idx_wait(1)
    gather_start(1, 1)
    scat_start(0, 0)

    @pl.loop(0, (NCHUNK - 5) // 4)
    def _quad(t):
      for j in range(4):
        q = (1 + j) & 3
        b = (1 + j) & 1
        q1, qm1 = (q + 1) & 3, (q - 1) & 3
        b1 = 1 - b
        cc = 1 + 4 * t + j
        gather_wait(q, b)
        scat_wait(qm1, b1)
        idx_wait(q1)
        gather_start(q1, b1)
        scat_start(q, b)
        idx_start(cc + 2, (q + 2) & 3)

    for cc in range(NCHUNK - 4, NCHUNK):
      q = cc & 3
      b = cc & 1
      q1, qm1 = (q + 1) & 3, (q - 1) & 3
      b1 = 1 - b
      gather_wait(q, b)
      scat_wait(qm1, b1)
      if cc + 1 < NCHUNK:
        idx_wait(q1)
        gather_start(q1, b1)
      scat_start(q, b)
      if cc + 2 < NCHUNK:
        idx_start(cc + 2, (q + 2) & 3)
    scat_wait((NCHUNK - 1) & 3, (NCHUNK - 1) & 1)

    plsc.subcore_barrier()

    pltpu.sync_copy(acc_sh.at[pl.ds(sid * RPT, RPT)],
                    acc_out.at[cid, pl.ds(sid * RPT, RPT)])
    if with_deg:
      pltpu.sync_copy(deg_sh.at[pl.ds(sid * RPT, RPT)],
                      deg_out.at[cid, pl.ds(sid * RPT, RPT)])

  run = pl.kernel(
      body, out_type=out_type, mesh=mesh, scratch_types=scratch,
      compiler_params=pltpu.CompilerParams(use_tc_tiling_on_sc=False))
  zrow = jnp.zeros((RPT, F), jnp.float32)
  if with_deg:
    res = run(table, src, dst, zrow,
              jnp.zeros((RPT, DCOL), jnp.float32),
              jnp.ones((CH, DCOL), jnp.float32))
  else:
    res = run(table, src, dst, zrow)
  return res if with_deg else res[0]


_RB = 1000
_GRID = N // _RB


def _row_spec():
  return pl.BlockSpec((_RB, F), lambda i: (i, 0))


def _full_spec():
  return pl.BlockSpec((F, F), lambda i: (0, 0))


def _deg_spec():
  return pl.BlockSpec((_RB, DCOL), lambda i: (i, 0))


def _pre_body(x_ref, wl_ref, wr_ref, b_ref, y_ref, r_ref):
  xb = x_ref[...]
  y_ref[...] = jnp.dot(xb, wl_ref[...], preferred_element_type=jnp.float32)
  r_ref[...] = (jnp.dot(xb, wr_ref[...], preferred_element_type=jnp.float32)
                + b_ref[...])


def _mid_body(p0, p1, d0, d1, r1, w2l, w2r, emb_ref, y2_ref, r2_ref):
  deg = jnp.maximum(d0[:, 0:1] + d1[:, 0:1], 1.0)
  emb = (p0[...] + p1[...]) / deg + r1[...]
  emb_ref[...] = emb
  h = jnp.maximum(emb, 0.0)
  y2_ref[...] = jnp.dot(h, w2l[...], preferred_element_type=jnp.float32)
  r2_ref[...] = jnp.dot(h, w2r[...], preferred_element_type=jnp.float32)


def _fin_body(q0, q1, d0, d1, r2, b2, out_ref):
  deg = jnp.maximum(d0[:, 0:1] + d1[:, 0:1], 1.0)
  out_ref[...] = (q0[...] + q1[...]) / deg + r2[...] + b2[...]


def kernel(x, edge_index, W1_l, b1_l, W1_r, W2_l, b2_l, W2_r):
  src = edge_index[0]
  dst = edge_index[1]
  f32 = jnp.float32

  pre = pl.pallas_call(
      _pre_body,
      grid=(_GRID,),
      in_specs=[_row_spec(), _full_spec(), _full_spec(),
                pl.BlockSpec((1, F), lambda i: (0, 0))],
      out_specs=[_row_spec(), _row_spec()],
      out_shape=[jax.ShapeDtypeStruct((N, F), f32)] * 2,
  )
  y1, r1 = pre(x, W1_l.T, W1_r.T, b1_l.reshape(1, F))

  acc1, degp = _sc_agg(y1, src, dst, with_deg=True)

  mid = pl.pallas_call(
      _mid_body,
      grid=(_GRID,),
      in_specs=[_row_spec(), _row_spec(), _deg_spec(), _deg_spec(),
                _row_spec(), _full_spec(), _full_spec()],
      out_specs=[_row_spec(), _row_spec(), _row_spec()],
      out_shape=[jax.ShapeDtypeStruct((N, F), f32)] * 3,
  )
  emb, y2, r2 = mid(acc1[0], acc1[1], degp[0], degp[1], r1,
                    W2_l.T, W2_r.T)

  acc2 = _sc_agg(y2, src, dst, with_deg=False)

  fin = pl.pallas_call(
      _fin_body,
      grid=(_GRID,),
      in_specs=[_row_spec(), _row_spec(), _deg_spec(), _deg_spec(),
                _row_spec(), pl.BlockSpec((1, F), lambda i: (0, 0))],
      out_specs=_row_spec(),
      out_shape=jax.ShapeDtypeStruct((N, F), f32),
  )
  logits = fin(acc2[0], acc2[1], degp[0], degp[1], r2, b2_l.reshape(1, F))

  return (logits, emb)
